# baseline (speedup 1.0000x reference)
import functools

import jax
import jax.numpy as jnp
from jax import lax
from jax.experimental import pallas as pl
from jax.experimental.pallas import tpu as pltpu
from jax.experimental.pallas import tpu_sc as plsc

B = 16
C = 3
IMG = 384
P = 16
G = IMG // P
N = G * G
E = 4
TOPK = 2
EXPERT_DIM = 1024
HIDDEN = 1024
PATCH_DIM = C * P * P


def _fuse_kernel(w1_ref, w2_ref, b1_ref, b2_ref, wc_ref, bc_ref):
    w2 = w2_ref[0]
    wc = jnp.dot(w1_ref[0], w2, preferred_element_type=jnp.float32)
    wc_ref[0] = wc.astype(jnp.bfloat16)
    bc_ref[0] = jnp.dot(b1_ref[0], w2, preferred_element_type=jnp.float32) + b2_ref[0]


def _routing_coeffs_sc(sel_t, rw_t):
    mesh = plsc.VectorSubcoreMesh(core_axis_name="c", subcore_axis_name="s")

    @functools.partial(
        pl.kernel,
        mesh=mesh,
        out_type=jax.ShapeDtypeStruct((E, B), jnp.float32),
        scratch_types=[
            pltpu.VMEM((TOPK, B), jnp.int32),
            pltpu.VMEM((TOPK, B), jnp.float32),
            pltpu.VMEM((E, B), jnp.float32),
        ],
    )
    def k(sel_hbm, rw_hbm, out_hbm, sel_v, rw_v, out_v):
        @pl.when((lax.axis_index("c") == 0) & (lax.axis_index("s") == 0))
        def _():
            pltpu.sync_copy(sel_hbm, sel_v)
            pltpu.sync_copy(rw_hbm, rw_v)
            s0 = sel_v[0]
            s1 = sel_v[1]
            w0 = rw_v[0]
            w1 = rw_v[1]
            zero = jnp.zeros((B,), jnp.float32)
            for e in range(E):
                out_v[e] = (jnp.where(s0 == e, w0, zero)
                            + jnp.where(s1 == e, w1, zero))
            pltpu.sync_copy(out_v, out_hbm)

    return k(sel_t, rw_t)


def _routed_matmul_kernel(wbe_ref, x_ref, wc_ref, bc_ref, out_ref):
    b = pl.program_id(0)
    cs = [wbe_ref[e, b] for e in range(E)]
    w_eff = cs[0].astype(jnp.bfloat16) * wc_ref[0]
    for e in range(1, E):
        w_eff = w_eff + cs[e].astype(jnp.bfloat16) * wc_ref[e]
    bias = cs[0] * bc_ref[0]
    for e in range(1, E):
        bias = bias + cs[e] * bc_ref[e]

    xb = x_ref[0].astype(jnp.bfloat16)
    patches = xb.reshape(C, G, P, G, P).transpose(1, 3, 0, 2, 4).reshape(
        N, PATCH_DIM)
    out_ref[0] = jnp.dot(patches, w_eff,
                         preferred_element_type=jnp.float32) + bias


def kernel(x, selected_experts, routing_weights, W_patch, b_patch, W_proj, b_proj):
    xb = x.shape[0]

    w_be = _routing_coeffs_sc(
        selected_experts.astype(jnp.int32).T, routing_weights.T)

    w_comb, b_comb = pl.pallas_call(
        _fuse_kernel,
        grid=(E,),
        in_specs=[
            pl.BlockSpec((1, PATCH_DIM, EXPERT_DIM), lambda e: (e, 0, 0)),
            pl.BlockSpec((1, EXPERT_DIM, HIDDEN), lambda e: (e, 0, 0)),
            pl.BlockSpec((1, 1, EXPERT_DIM), lambda e: (e, 0, 0)),
            pl.BlockSpec((1, 1, HIDDEN), lambda e: (e, 0, 0)),
        ],
        out_specs=[
            pl.BlockSpec((1, PATCH_DIM, HIDDEN), lambda e: (e, 0, 0)),
            pl.BlockSpec((1, 1, HIDDEN), lambda e: (e, 0, 0)),
        ],
        out_shape=[
            jax.ShapeDtypeStruct((E, PATCH_DIM, HIDDEN), jnp.bfloat16),
            jax.ShapeDtypeStruct((E, 1, HIDDEN), jnp.float32),
        ],
    )(W_patch, W_proj, b_patch.reshape(E, 1, EXPERT_DIM),
      b_proj.reshape(E, 1, HIDDEN))

    out = pl.pallas_call(
        _routed_matmul_kernel,
        grid_spec=pltpu.PrefetchScalarGridSpec(
            num_scalar_prefetch=1,
            grid=(xb,),
            in_specs=[
                pl.BlockSpec((1, C, IMG, IMG), lambda b, wbe: (b, 0, 0, 0)),
                pl.BlockSpec((E, PATCH_DIM, HIDDEN), lambda b, wbe: (0, 0, 0)),
                pl.BlockSpec((E, 1, HIDDEN), lambda b, wbe: (0, 0, 0)),
            ],
            out_specs=pl.BlockSpec((1, N, HIDDEN), lambda b, wbe: (b, 0, 0)),
        ),
        out_shape=jax.ShapeDtypeStruct((xb, N, HIDDEN), jnp.float32),
    )(w_be, x, w_comb, b_comb)
    return out


# speedup vs baseline: 1.0954x; 1.0954x over previous
import jax
import jax.numpy as jnp
from jax.experimental import pallas as pl
from jax.experimental.pallas import tpu as pltpu

B = 16
C = 3
IMG = 384
P = 16
G = IMG // P
N = G * G
E = 4
TOPK = 2
EXPERT_DIM = 1024
HIDDEN = 1024
PATCH_DIM = C * P * P


def _fuse_kernel(w1_ref, w2_ref, b1_ref, b2_ref, wc_ref, bc_ref):
    w2 = w2_ref[0]
    wc = jnp.dot(w1_ref[0], w2, preferred_element_type=jnp.float32)
    wc_ref[0] = wc.astype(jnp.bfloat16)
    bc_ref[0] = jnp.dot(b1_ref[0], w2, preferred_element_type=jnp.float32) + b2_ref[0]


def _routed_matmul_kernel(sel_ref, rw_ref, x_ref, wc_ref, bc_ref, out_ref):
    b = pl.program_id(0)
    s0 = sel_ref[b, 0]
    s1 = sel_ref[b, 1]
    w0 = rw_ref[b, 0]
    w1 = rw_ref[b, 1]

    def coef(e):
        c0 = jnp.where(s0 == e, w0, jnp.float32(0.0))
        c1 = jnp.where(s1 == e, w1, jnp.float32(0.0))
        return c0 + c1

    cs = [coef(e) for e in range(E)]
    w_eff = cs[0].astype(jnp.bfloat16) * wc_ref[0]
    for e in range(1, E):
        w_eff = w_eff + cs[e].astype(jnp.bfloat16) * wc_ref[e]
    bias = cs[0] * bc_ref[0]
    for e in range(1, E):
        bias = bias + cs[e] * bc_ref[e]

    xb = x_ref[0].astype(jnp.bfloat16)
    patches = xb.reshape(C, G, P, G, P).transpose(1, 3, 0, 2, 4).reshape(
        N, PATCH_DIM)
    out_ref[0] = jnp.dot(patches, w_eff,
                         preferred_element_type=jnp.float32) + bias


def kernel(x, selected_experts, routing_weights, W_patch, b_patch, W_proj, b_proj):
    xb = x.shape[0]

    w_comb, b_comb = pl.pallas_call(
        _fuse_kernel,
        grid=(E,),
        in_specs=[
            pl.BlockSpec((1, PATCH_DIM, EXPERT_DIM), lambda e: (e, 0, 0)),
            pl.BlockSpec((1, EXPERT_DIM, HIDDEN), lambda e: (e, 0, 0)),
            pl.BlockSpec((1, 1, EXPERT_DIM), lambda e: (e, 0, 0)),
            pl.BlockSpec((1, 1, HIDDEN), lambda e: (e, 0, 0)),
        ],
        out_specs=[
            pl.BlockSpec((1, PATCH_DIM, HIDDEN), lambda e: (e, 0, 0)),
            pl.BlockSpec((1, 1, HIDDEN), lambda e: (e, 0, 0)),
        ],
        out_shape=[
            jax.ShapeDtypeStruct((E, PATCH_DIM, HIDDEN), jnp.bfloat16),
            jax.ShapeDtypeStruct((E, 1, HIDDEN), jnp.float32),
        ],
    )(W_patch, W_proj, b_patch.reshape(E, 1, EXPERT_DIM),
      b_proj.reshape(E, 1, HIDDEN))

    out = pl.pallas_call(
        _routed_matmul_kernel,
        grid_spec=pltpu.PrefetchScalarGridSpec(
            num_scalar_prefetch=2,
            grid=(xb,),
            in_specs=[
                pl.BlockSpec((1, C, IMG, IMG), lambda b, sel, rw: (b, 0, 0, 0)),
                pl.BlockSpec((E, PATCH_DIM, HIDDEN), lambda b, sel, rw: (0, 0, 0)),
                pl.BlockSpec((E, 1, HIDDEN), lambda b, sel, rw: (0, 0, 0)),
            ],
            out_specs=pl.BlockSpec((1, N, HIDDEN), lambda b, sel, rw: (b, 0, 0)),
        ),
        out_shape=jax.ShapeDtypeStruct((xb, N, HIDDEN), jnp.float32),
    )(selected_experts.astype(jnp.int32), routing_weights, x, w_comb,
      b_comb)
    return out
